# baseline (speedup 1.0000x reference)
import functools

import jax
import jax.numpy as jnp
from jax import lax
from jax.experimental import pallas as pl
from jax.experimental.pallas import tpu as pltpu
from jax.experimental.pallas import tpu_sc as plsc

_H = 128
_NB = 256
_EB = 512
_CH = 256
_SC_CH = 128
_NC = 2
_NS = 16


def _rup(n, m):
    return (n + m - 1) // m * m


def _ln(h, g, be):
    mu = jnp.mean(h, axis=1, keepdims=True)
    xc = h - mu
    var = jnp.mean(xc * xc, axis=1, keepdims=True)
    return xc * lax.rsqrt(var + 1e-5) * g + be


def _dot(a, b):
    return jnp.dot(a.astype(jnp.bfloat16), b.astype(jnp.bfloat16),
                   preferred_element_type=jnp.float32)


def _make_gather(B, H):
    NW = _NC * _NS
    rows_pw = B // NW
    nch = rows_pw // _SC_CH
    assert rows_pw * NW == B and nch * _SC_CH == rows_pw and nch % 2 == 0
    mesh = plsc.VectorSubcoreMesh(core_axis_name="c", subcore_axis_name="s")

    @functools.partial(
        pl.kernel,
        out_type=jax.ShapeDtypeStruct((B, H), jnp.float32),
        mesh=mesh,
        scratch_types=[
            pltpu.VMEM((rows_pw,), jnp.int32),
            pltpu.VMEM((_SC_CH, H), jnp.float32),
            pltpu.VMEM((_SC_CH, H), jnp.float32),
            pltpu.SemaphoreType.DMA,
            pltpu.SemaphoreType.DMA,
            pltpu.SemaphoreType.DMA,
        ],
    )
    def gat(table_hbm, idx_hbm, out_hbm, idx_v, rows0, rows1, sem_g,
            sem_o0, sem_o1):
        wid = lax.axis_index("s") * _NC + lax.axis_index("c")
        wbase = pl.multiple_of(wid * rows_pw, _SC_CH)
        pltpu.sync_copy(idx_hbm.at[pl.ds(wbase, rows_pw)], idx_v)
        bufs = ((rows0, sem_o0), (rows1, sem_o1))

        def body(j, carry):
            for b in range(2):
                c = j * 2 + b
                off = c * _SC_CH
                rows_v, sem_o = bufs[b]
                dst = out_hbm.at[pl.ds(wbase + off, _SC_CH)]

                @pl.when(j > 0)
                def _drain():
                    pltpu.make_async_copy(rows_v, dst, sem_o).wait()

                g = pltpu.make_async_copy(
                    table_hbm.at[idx_v.at[pl.ds(off, _SC_CH)]], rows_v, sem_g)
                g.start()
                g.wait()
                pltpu.make_async_copy(rows_v, dst, sem_o).start()
            return carry

        lax.fori_loop(0, nch // 2, body, 0)
        for b in range(2):
            rows_v, sem_o = bufs[b]
            pltpu.make_async_copy(
                rows_v, out_hbm.at[pl.ds(wbase, _SC_CH)], sem_o).wait()

    return gat


def _enc_body(x_ref, w1_ref, b1_ref, w2_ref, b2_ref, w3_ref, b3_ref,
              g_ref, be_ref, o_ref):
    h = jnp.maximum(_dot(x_ref[...], w1_ref[...]) + b1_ref[...], 0.0)
    h = jnp.maximum(_dot(h, w2_ref[...]) + b2_ref[...], 0.0)
    h = _dot(h, w3_ref[...]) + b3_ref[...]
    o_ref[...] = _ln(h, g_ref[...], be_ref[...])


def _mp_body(offs_ref, nh_ref, eh_any, gs_any, dst_any,
             ew1_ref, eb1_ref, ew2_ref, eb2_ref, ew3_ref, eb3_ref,
             eg_ref, ebe_ref,
             nw1_ref, nb1_ref, nw2_ref, nb2_ref, nw3_ref, nb3_ref,
             ng_ref, nbe_ref,
             o_node_ref, o_edge_any,
             ebuf, sbuf, dbuf, obuf, sem_in0, sem_in1, sem_o0, sem_o1):
    i = pl.program_id(0)
    start = offs_ref[i]
    end = offs_ref[i + 1]
    nck = (end - start + (_CH - 1)) // _CH
    npair = (nck + 1) // 2
    ntot = 2 * npair
    nh = nh_ref[...]
    cols = i * _NB + lax.broadcasted_iota(jnp.int32, (1, _NB), 1)
    sem_in = (sem_in0, sem_in1)
    sem_o = (sem_o0, sem_o1)
    ew1 = ew1_ref[...]

    def in_copies(c, b):
        base = start + c * _CH
        return (pltpu.make_async_copy(eh_any.at[pl.ds(base, _CH), :],
                                      ebuf.at[b], sem_in[b]),
                pltpu.make_async_copy(gs_any.at[pl.ds(base, _CH), :],
                                      sbuf.at[b], sem_in[b]),
                pltpu.make_async_copy(dst_any.at[pl.ds(base, _CH), :],
                                      dbuf.at[b], sem_in[b]))

    def prefetch(c, b):
        for cp in in_copies(c, b):
            cp.start()

    @pl.when(nck > 0)
    def _prime():
        prefetch(0, 0)
        prefetch(1, 1)

    def chunk(c, b, agg):
        base = start + c * _CH
        for cp in in_copies(c, b):
            cp.wait()
        e = ebuf[b]
        s = sbuf[b]
        dv = dbuf[b]

        @pl.when(c + 2 < ntot)
        def _next():
            prefetch(c + 2, b)

        ids = base + lax.broadcasted_iota(jnp.int32, (_CH, 1), 0)
        oh = jnp.where((dv == cols) & (ids < end), 1.0, 0.0)
        nd = _dot(oh, nh)
        h = (_dot(e, ew1[0:_H]) + _dot(s, ew1[_H:2 * _H])
             + _dot(nd, ew1[2 * _H:]) + eb1_ref[...])
        h = jnp.maximum(h, 0.0)
        h = jnp.maximum(_dot(h, ew2_ref[...]) + eb2_ref[...], 0.0)
        h = _dot(h, ew3_ref[...]) + eb3_ref[...]
        enew = _ln(h, eg_ref[...], ebe_ref[...]) + e
        out_cp = pltpu.make_async_copy(
            obuf.at[b], o_edge_any.at[pl.ds(base, _CH), :], sem_o[b])

        @pl.when(c >= 2)
        def _drain():
            out_cp.wait()

        obuf[b] = enew
        out_cp.start()
        return agg + lax.dot_general(
            oh.astype(jnp.bfloat16), enew.astype(jnp.bfloat16),
            (((0,), (0,)), ((), ())), preferred_element_type=jnp.float32)

    def pair(j, agg):
        agg = chunk(2 * j, 0, agg)
        return chunk(2 * j + 1, 1, agg)

    agg = lax.fori_loop(0, npair, pair, jnp.zeros((_NB, _H), jnp.float32))

    @pl.when(nck > 0)
    def _final_drain():
        for b in range(2):
            pltpu.make_async_copy(
                obuf.at[b], o_edge_any.at[pl.ds(start, _CH), :],
                sem_o[b]).wait()

    nw1 = nw1_ref[...]
    h = jnp.maximum(_dot(nh, nw1[:_H]) + _dot(agg, nw1[_H:]) + nb1_ref[...],
                    0.0)
    h = jnp.maximum(_dot(h, nw2_ref[...]) + nb2_ref[...], 0.0)
    h = _dot(h, nw3_ref[...]) + nb3_ref[...]
    o_node_ref[...] = _ln(h, ng_ref[...], nbe_ref[...]) + nh


def _dec_body(nh_ref, w1_ref, b1_ref, w2_ref, b2_ref, w3_ref, b3_ref,
              xr_ref, o_ref):
    h = jnp.maximum(_dot(nh_ref[...], w1_ref[...]) + b1_ref[...], 0.0)
    h = jnp.maximum(_dot(h, w2_ref[...]) + b2_ref[...], 0.0)
    o_ref[...] = _dot(h, w3_ref[...]) + b3_ref[...] + xr_ref[...]


def _wspec(shape):
    return pl.BlockSpec(shape, lambda i: tuple(0 for _ in shape))


def _enc_call(xin, W1, b1, W2, b2, W3, b3, g, be, blk):
    R = xin.shape[0]
    return pl.pallas_call(
        _enc_body,
        grid=(R // blk,),
        in_specs=[pl.BlockSpec((blk, xin.shape[1]), lambda i: (i, 0)),
                  _wspec(W1.shape), _wspec((1, _H)), _wspec(W2.shape),
                  _wspec((1, _H)), _wspec(W3.shape), _wspec((1, _H)),
                  _wspec((1, _H)), _wspec((1, _H))],
        out_specs=pl.BlockSpec((blk, _H), lambda i: (i, 0)),
        out_shape=jax.ShapeDtypeStruct((R, _H), jnp.float32),
    )(xin, W1, b1, W2, b2, W3, b3, g, be)


def _mp_call(node_h, offs, edge_h, g_src, dst2d, ew, nw):
    N_pad = node_h.shape[0]
    E_pad = edge_h.shape[0]
    return pl.pallas_call(
        _mp_body,
        grid=(N_pad // _NB,),
        in_specs=[pl.BlockSpec(memory_space=pltpu.SMEM),
                  pl.BlockSpec((_NB, _H), lambda i: (i, 0)),
                  pl.BlockSpec(memory_space=pl.ANY),
                  pl.BlockSpec(memory_space=pl.ANY),
                  pl.BlockSpec(memory_space=pl.ANY),
                  _wspec(ew[0].shape), _wspec((1, _H)), _wspec(ew[2].shape),
                  _wspec((1, _H)), _wspec(ew[4].shape), _wspec((1, _H)),
                  _wspec((1, _H)), _wspec((1, _H)),
                  _wspec(nw[0].shape), _wspec((1, _H)), _wspec(nw[2].shape),
                  _wspec((1, _H)), _wspec(nw[4].shape), _wspec((1, _H)),
                  _wspec((1, _H)), _wspec((1, _H))],
        out_specs=[pl.BlockSpec((_NB, _H), lambda i: (i, 0)),
                   pl.BlockSpec(memory_space=pl.ANY)],
        out_shape=[jax.ShapeDtypeStruct((N_pad, _H), jnp.float32),
                   jax.ShapeDtypeStruct((E_pad, _H), jnp.float32)],
        scratch_shapes=[pltpu.VMEM((2, _CH, _H), jnp.float32),
                        pltpu.VMEM((2, _CH, _H), jnp.float32),
                        pltpu.VMEM((2, _CH, 1), jnp.int32),
                        pltpu.VMEM((2, _CH, _H), jnp.float32),
                        pltpu.SemaphoreType.DMA,
                        pltpu.SemaphoreType.DMA,
                        pltpu.SemaphoreType.DMA,
                        pltpu.SemaphoreType.DMA],
    )(offs, node_h, edge_h, g_src, dst2d, *ew, *nw)


def _dec_call(node_h, W1, b1, W2, b2, W3, b3, xres):
    N_pad = node_h.shape[0]
    return pl.pallas_call(
        _dec_body,
        grid=(N_pad // _NB,),
        in_specs=[pl.BlockSpec((_NB, _H), lambda i: (i, 0)),
                  _wspec(W1.shape), _wspec((1, _H)), _wspec(W2.shape),
                  _wspec((1, _H)), _wspec(W3.shape), _wspec((1, _H)),
                  pl.BlockSpec((_NB, _H), lambda i: (i, 0))],
        out_specs=pl.BlockSpec((_NB, _H), lambda i: (i, 0)),
        out_shape=jax.ShapeDtypeStruct((N_pad, _H), jnp.float32),
    )(node_h, W1, b1, W2, b2, W3, b3, xres)


def _prep3(p, in_pad=None, out_pad=None):
    (W1, b1), (W2, b2), (W3, b3) = p["lin"]
    if in_pad is not None and W1.shape[0] < in_pad:
        W1 = jnp.zeros((in_pad, W1.shape[1]), jnp.float32).at[:W1.shape[0]].set(W1)
    if out_pad is not None and W3.shape[1] < out_pad:
        W3 = jnp.zeros((W3.shape[0], out_pad), jnp.float32).at[:, :W3.shape[1]].set(W3)
        b3 = jnp.zeros((out_pad,), jnp.float32).at[:b3.shape[0]].set(b3)
    ws = [W1, b1.reshape(1, -1), W2, b2.reshape(1, -1), W3, b3.reshape(1, -1)]
    if "ln" in p:
        g, be = p["ln"]
        ws += [g.reshape(1, -1), be.reshape(1, -1)]
    return ws


def kernel(x, edge_index, edge_attr, node_type, params):
    N, ndim = x.shape
    E, e_in = edge_attr.shape
    N_pad = _rup(N, _NB)
    E_pad = _rup(E + 2 * _CH, _NC * _NS * _SC_CH)
    nblk = N_pad // _NB

    src = edge_index[0].astype(jnp.int32)
    dst = edge_index[1].astype(jnp.int32)
    perm = jnp.argsort(dst)
    dst_s = dst[perm]
    src_s = src[perm]
    pad_e = E_pad - E
    dst_sp = jnp.concatenate([dst_s, jnp.full((pad_e,), N_pad - 1, jnp.int32)])
    src_sp = jnp.concatenate([src_s, jnp.zeros((pad_e,), jnp.int32)])
    perm_p = jnp.concatenate([perm.astype(jnp.int32),
                              jnp.zeros((pad_e,), jnp.int32)])
    offs = jnp.searchsorted(
        dst_s, jnp.arange(nblk + 1, dtype=jnp.int32) * _NB).astype(jnp.int32)
    dst2d = dst_sp.reshape(E_pad, 1)

    nt = jnp.squeeze(node_type).astype(jnp.int32)
    onehot = jax.nn.one_hot(nt, 2, dtype=jnp.float32)
    xin = (jnp.zeros((N_pad, _H), jnp.float32)
           .at[:N, :ndim].set(x).at[:N, ndim:ndim + 2].set(onehot))
    ein = jnp.zeros((E_pad, _H), jnp.float32).at[:E, :e_in].set(edge_attr)
    xres = jnp.zeros((N_pad, _H), jnp.float32).at[:N, :ndim].set(x)

    enc_n = _prep3(params["node_enc"], in_pad=_H)
    enc_e = _prep3(params["edge_enc"], in_pad=_H)
    dec_w = _prep3(params["dec"], out_pad=_H)

    node_h = _enc_call(xin, *enc_n, blk=_NB)
    edge_h_u = _enc_call(ein, *enc_e, blk=_EB)

    gat_E = _make_gather(E_pad, _H)
    edge_h = gat_E(edge_h_u, perm_p)

    for blk in params["mp"]:
        g_src = gat_E(node_h, src_sp)
        node_h, edge_h = _mp_call(node_h, offs, edge_h, g_src, dst2d,
                                  _prep3(blk["edge"]), _prep3(blk["node"]))

    out = _dec_call(node_h, *dec_w, xres)
    return out[:N, :ndim]


# speedup vs baseline: 1.3538x; 1.3538x over previous
import functools

import jax
import jax.numpy as jnp
from jax import lax
from jax.experimental import pallas as pl
from jax.experimental.pallas import tpu as pltpu
from jax.experimental.pallas import tpu_sc as plsc

_H = 128
_NB = 256
_EB = 512
_CH = 512
_SC_CH = 128
_NC = 2
_NS = 16


def _rup(n, m):
    return (n + m - 1) // m * m


def _ln(h, g, be):
    mu = jnp.mean(h, axis=1, keepdims=True)
    xc = h - mu
    var = jnp.mean(xc * xc, axis=1, keepdims=True)
    return xc * lax.rsqrt(var + 1e-5) * g + be


def _dot(a, b):
    return jnp.dot(a, b, preferred_element_type=jnp.float32)


def _make_gather(B, H):
    NW = _NC * _NS
    rows_pw = B // NW
    nch = rows_pw // _SC_CH
    assert rows_pw * NW == B and nch * _SC_CH == rows_pw and nch % 2 == 0
    mesh = plsc.VectorSubcoreMesh(core_axis_name="c", subcore_axis_name="s")

    @functools.partial(
        pl.kernel,
        out_type=jax.ShapeDtypeStruct((B, H), jnp.float32),
        mesh=mesh,
        scratch_types=[
            pltpu.VMEM((rows_pw,), jnp.int32),
            pltpu.VMEM((_SC_CH, H), jnp.float32),
            pltpu.VMEM((_SC_CH, H), jnp.float32),
            pltpu.SemaphoreType.DMA,
            pltpu.SemaphoreType.DMA,
            pltpu.SemaphoreType.DMA,
        ],
    )
    def gat(table_hbm, idx_hbm, out_hbm, idx_v, rows0, rows1, sem_g,
            sem_o0, sem_o1):
        wid = lax.axis_index("s") * _NC + lax.axis_index("c")
        wbase = pl.multiple_of(wid * rows_pw, _SC_CH)
        pltpu.sync_copy(idx_hbm.at[pl.ds(wbase, rows_pw)], idx_v)
        bufs = ((rows0, sem_o0), (rows1, sem_o1))

        def body(j, carry):
            for b in range(2):
                c = j * 2 + b
                off = c * _SC_CH
                rows_v, sem_o = bufs[b]
                dst = out_hbm.at[pl.ds(wbase + off, _SC_CH)]

                @pl.when(j > 0)
                def _drain():
                    pltpu.make_async_copy(rows_v, dst, sem_o).wait()

                g = pltpu.make_async_copy(
                    table_hbm.at[idx_v.at[pl.ds(off, _SC_CH)]], rows_v, sem_g)
                g.start()
                g.wait()
                pltpu.make_async_copy(rows_v, dst, sem_o).start()
            return carry

        lax.fori_loop(0, nch // 2, body, 0)
        for b in range(2):
            rows_v, sem_o = bufs[b]
            pltpu.make_async_copy(
                rows_v, out_hbm.at[pl.ds(wbase, _SC_CH)], sem_o).wait()

    return gat


def _enc_body(x_ref, w1_ref, b1_ref, w2_ref, b2_ref, w3_ref, b3_ref,
              g_ref, be_ref, o_ref):
    h = jnp.maximum(_dot(x_ref[...], w1_ref[...]) + b1_ref[...], 0.0)
    h = jnp.maximum(_dot(h, w2_ref[...]) + b2_ref[...], 0.0)
    h = _dot(h, w3_ref[...]) + b3_ref[...]
    o_ref[...] = _ln(h, g_ref[...], be_ref[...])


def _mp_body(offs_ref, nh_ref, eh_any, gs_any, dst_any,
             ew1_ref, eb1_ref, ew2_ref, eb2_ref, ew3_ref, eb3_ref,
             eg_ref, ebe_ref,
             nw1_ref, nb1_ref, nw2_ref, nb2_ref, nw3_ref, nb3_ref,
             ng_ref, nbe_ref,
             o_node_ref, o_edge_any,
             ebuf, sbuf, dbuf, obuf, sem_in0, sem_in1, sem_o0, sem_o1):
    i = pl.program_id(0)
    start = offs_ref[i]
    end = offs_ref[i + 1]
    nck = (end - start + (_CH - 1)) // _CH
    npair = (nck + 1) // 2
    ntot = 2 * npair
    nh = nh_ref[...]
    cols = i * _NB + lax.broadcasted_iota(jnp.int32, (1, _NB), 1)
    sem_in = (sem_in0, sem_in1)
    sem_o = (sem_o0, sem_o1)
    ew1 = ew1_ref[...]

    def in_copies(c, b):
        base = start + c * _CH
        return (pltpu.make_async_copy(eh_any.at[pl.ds(base, _CH), :],
                                      ebuf.at[b], sem_in[b]),
                pltpu.make_async_copy(gs_any.at[pl.ds(base, _CH), :],
                                      sbuf.at[b], sem_in[b]),
                pltpu.make_async_copy(dst_any.at[pl.ds(base, _CH), :],
                                      dbuf.at[b], sem_in[b]))

    def prefetch(c, b):
        for cp in in_copies(c, b):
            cp.start()

    @pl.when(nck > 0)
    def _prime():
        prefetch(0, 0)
        prefetch(1, 1)

    def chunk(c, b, agg):
        base = start + c * _CH
        for cp in in_copies(c, b):
            cp.wait()
        e = ebuf[b]
        s = sbuf[b]
        dv = dbuf[b]

        @pl.when(c + 2 < ntot)
        def _next():
            prefetch(c + 2, b)

        ids = base + lax.broadcasted_iota(jnp.int32, (_CH, 1), 0)
        oh = jnp.where((dv == cols) & (ids < end), 1.0, 0.0)
        nd = _dot(oh, nh)
        h = (_dot(e, ew1[0:_H]) + _dot(s, ew1[_H:2 * _H])
             + _dot(nd, ew1[2 * _H:]) + eb1_ref[...])
        h = jnp.maximum(h, 0.0)
        h = jnp.maximum(_dot(h, ew2_ref[...]) + eb2_ref[...], 0.0)
        h = _dot(h, ew3_ref[...]) + eb3_ref[...]
        enew = _ln(h, eg_ref[...], ebe_ref[...]) + e
        out_cp = pltpu.make_async_copy(
            obuf.at[b], o_edge_any.at[pl.ds(base, _CH), :], sem_o[b])

        @pl.when(c >= 2)
        def _drain():
            out_cp.wait()

        obuf[b] = enew
        out_cp.start()
        return agg + lax.dot_general(oh, enew, (((0,), (0,)), ((), ())),
                                     preferred_element_type=jnp.float32)

    def pair(j, agg):
        agg = chunk(2 * j, 0, agg)
        return chunk(2 * j + 1, 1, agg)

    agg = lax.fori_loop(0, npair, pair, jnp.zeros((_NB, _H), jnp.float32))

    @pl.when(nck > 0)
    def _final_drain():
        for b in range(2):
            pltpu.make_async_copy(
                obuf.at[b], o_edge_any.at[pl.ds(start, _CH), :],
                sem_o[b]).wait()

    nw1 = nw1_ref[...]
    h = jnp.maximum(_dot(nh, nw1[:_H]) + _dot(agg, nw1[_H:]) + nb1_ref[...],
                    0.0)
    h = jnp.maximum(_dot(h, nw2_ref[...]) + nb2_ref[...], 0.0)
    h = _dot(h, nw3_ref[...]) + nb3_ref[...]
    o_node_ref[...] = _ln(h, ng_ref[...], nbe_ref[...]) + nh


def _dec_body(nh_ref, w1_ref, b1_ref, w2_ref, b2_ref, w3_ref, b3_ref,
              xr_ref, o_ref):
    h = jnp.maximum(_dot(nh_ref[...], w1_ref[...]) + b1_ref[...], 0.0)
    h = jnp.maximum(_dot(h, w2_ref[...]) + b2_ref[...], 0.0)
    o_ref[...] = _dot(h, w3_ref[...]) + b3_ref[...] + xr_ref[...]


def _wspec(shape):
    return pl.BlockSpec(shape, lambda i: tuple(0 for _ in shape))


def _enc_call(xin, W1, b1, W2, b2, W3, b3, g, be, blk):
    R = xin.shape[0]
    return pl.pallas_call(
        _enc_body,
        grid=(R // blk,),
        in_specs=[pl.BlockSpec((blk, xin.shape[1]), lambda i: (i, 0)),
                  _wspec(W1.shape), _wspec((1, _H)), _wspec(W2.shape),
                  _wspec((1, _H)), _wspec(W3.shape), _wspec((1, _H)),
                  _wspec((1, _H)), _wspec((1, _H))],
        out_specs=pl.BlockSpec((blk, _H), lambda i: (i, 0)),
        out_shape=jax.ShapeDtypeStruct((R, _H), jnp.float32),
    )(xin, W1, b1, W2, b2, W3, b3, g, be)


def _mp_call(node_h, offs, edge_h, g_src, dst2d, ew, nw):
    N_pad = node_h.shape[0]
    E_pad = edge_h.shape[0]
    return pl.pallas_call(
        _mp_body,
        grid=(N_pad // _NB,),
        in_specs=[pl.BlockSpec(memory_space=pltpu.SMEM),
                  pl.BlockSpec((_NB, _H), lambda i: (i, 0)),
                  pl.BlockSpec(memory_space=pl.ANY),
                  pl.BlockSpec(memory_space=pl.ANY),
                  pl.BlockSpec(memory_space=pl.ANY),
                  _wspec(ew[0].shape), _wspec((1, _H)), _wspec(ew[2].shape),
                  _wspec((1, _H)), _wspec(ew[4].shape), _wspec((1, _H)),
                  _wspec((1, _H)), _wspec((1, _H)),
                  _wspec(nw[0].shape), _wspec((1, _H)), _wspec(nw[2].shape),
                  _wspec((1, _H)), _wspec(nw[4].shape), _wspec((1, _H)),
                  _wspec((1, _H)), _wspec((1, _H))],
        out_specs=[pl.BlockSpec((_NB, _H), lambda i: (i, 0)),
                   pl.BlockSpec(memory_space=pl.ANY)],
        out_shape=[jax.ShapeDtypeStruct((N_pad, _H), jnp.float32),
                   jax.ShapeDtypeStruct((E_pad, _H), jnp.float32)],
        scratch_shapes=[pltpu.VMEM((2, _CH, _H), jnp.float32),
                        pltpu.VMEM((2, _CH, _H), jnp.float32),
                        pltpu.VMEM((2, _CH, 1), jnp.int32),
                        pltpu.VMEM((2, _CH, _H), jnp.float32),
                        pltpu.SemaphoreType.DMA,
                        pltpu.SemaphoreType.DMA,
                        pltpu.SemaphoreType.DMA,
                        pltpu.SemaphoreType.DMA],
    )(offs, node_h, edge_h, g_src, dst2d, *ew, *nw)


def _dec_call(node_h, W1, b1, W2, b2, W3, b3, xres):
    N_pad = node_h.shape[0]
    return pl.pallas_call(
        _dec_body,
        grid=(N_pad // _NB,),
        in_specs=[pl.BlockSpec((_NB, _H), lambda i: (i, 0)),
                  _wspec(W1.shape), _wspec((1, _H)), _wspec(W2.shape),
                  _wspec((1, _H)), _wspec(W3.shape), _wspec((1, _H)),
                  pl.BlockSpec((_NB, _H), lambda i: (i, 0))],
        out_specs=pl.BlockSpec((_NB, _H), lambda i: (i, 0)),
        out_shape=jax.ShapeDtypeStruct((N_pad, _H), jnp.float32),
    )(node_h, W1, b1, W2, b2, W3, b3, xres)


def _prep3(p, in_pad=None, out_pad=None):
    (W1, b1), (W2, b2), (W3, b3) = p["lin"]
    if in_pad is not None and W1.shape[0] < in_pad:
        W1 = jnp.zeros((in_pad, W1.shape[1]), jnp.float32).at[:W1.shape[0]].set(W1)
    if out_pad is not None and W3.shape[1] < out_pad:
        W3 = jnp.zeros((W3.shape[0], out_pad), jnp.float32).at[:, :W3.shape[1]].set(W3)
        b3 = jnp.zeros((out_pad,), jnp.float32).at[:b3.shape[0]].set(b3)
    ws = [W1, b1.reshape(1, -1), W2, b2.reshape(1, -1), W3, b3.reshape(1, -1)]
    if "ln" in p:
        g, be = p["ln"]
        ws += [g.reshape(1, -1), be.reshape(1, -1)]
    return ws


def kernel(x, edge_index, edge_attr, node_type, params):
    N, ndim = x.shape
    E, e_in = edge_attr.shape
    N_pad = _rup(N, _NB)
    E_pad = _rup(E + 2 * _CH, _NC * _NS * _SC_CH)
    nblk = N_pad // _NB

    src = edge_index[0].astype(jnp.int32)
    dst = edge_index[1].astype(jnp.int32)
    perm = jnp.argsort(dst)
    dst_s = dst[perm]
    src_s = src[perm]
    pad_e = E_pad - E
    dst_sp = jnp.concatenate([dst_s, jnp.full((pad_e,), N_pad - 1, jnp.int32)])
    src_sp = jnp.concatenate([src_s, jnp.zeros((pad_e,), jnp.int32)])
    perm_p = jnp.concatenate([perm.astype(jnp.int32),
                              jnp.zeros((pad_e,), jnp.int32)])
    offs = jnp.searchsorted(
        dst_s, jnp.arange(nblk + 1, dtype=jnp.int32) * _NB).astype(jnp.int32)
    dst2d = dst_sp.reshape(E_pad, 1)

    nt = jnp.squeeze(node_type).astype(jnp.int32)
    onehot = jax.nn.one_hot(nt, 2, dtype=jnp.float32)
    xin = (jnp.zeros((N_pad, _H), jnp.float32)
           .at[:N, :ndim].set(x).at[:N, ndim:ndim + 2].set(onehot))
    ein = jnp.zeros((E_pad, _H), jnp.float32).at[:E, :e_in].set(edge_attr)
    xres = jnp.zeros((N_pad, _H), jnp.float32).at[:N, :ndim].set(x)

    enc_n = _prep3(params["node_enc"], in_pad=_H)
    enc_e = _prep3(params["edge_enc"], in_pad=_H)
    dec_w = _prep3(params["dec"], out_pad=_H)

    node_h = _enc_call(xin, *enc_n, blk=_NB)
    edge_h_u = _enc_call(ein, *enc_e, blk=_EB)

    gat_E = _make_gather(E_pad, _H)
    edge_h = gat_E(edge_h_u, perm_p)

    for blk in params["mp"]:
        g_src = gat_E(node_h, src_sp)
        node_h, edge_h = _mp_call(node_h, offs, edge_h, g_src, dst2d,
                                  _prep3(blk["edge"]), _prep3(blk["node"]))

    out = _dec_call(node_h, *dec_w, xres)
    return out[:N, :ndim]


# speedup vs baseline: 1.5520x; 1.1464x over previous
import functools

import jax
import jax.numpy as jnp
from jax import lax
from jax.experimental import pallas as pl
from jax.experimental.pallas import tpu as pltpu
from jax.experimental.pallas import tpu_sc as plsc

_H = 128
_NB = 256
_EB = 512
_CH = 1024
_SC_CH = 128
_NC = 2
_NS = 16


def _rup(n, m):
    return (n + m - 1) // m * m


def _ln(h, g, be):
    mu = jnp.mean(h, axis=1, keepdims=True)
    xc = h - mu
    var = jnp.mean(xc * xc, axis=1, keepdims=True)
    return xc * lax.rsqrt(var + 1e-5) * g + be


def _dot(a, b):
    return jnp.dot(a, b, preferred_element_type=jnp.float32)


def _make_gather(B, H):
    NW = _NC * _NS
    rows_pw = B // NW
    nch = rows_pw // _SC_CH
    assert rows_pw * NW == B and nch * _SC_CH == rows_pw and nch % 2 == 0
    mesh = plsc.VectorSubcoreMesh(core_axis_name="c", subcore_axis_name="s")

    @functools.partial(
        pl.kernel,
        out_type=jax.ShapeDtypeStruct((B, H), jnp.float32),
        mesh=mesh,
        scratch_types=[
            pltpu.VMEM((rows_pw,), jnp.int32),
            pltpu.VMEM((_SC_CH, H), jnp.float32),
            pltpu.VMEM((_SC_CH, H), jnp.float32),
            pltpu.SemaphoreType.DMA,
            pltpu.SemaphoreType.DMA,
            pltpu.SemaphoreType.DMA,
        ],
    )
    def gat(table_hbm, idx_hbm, out_hbm, idx_v, rows0, rows1, sem_g,
            sem_o0, sem_o1):
        wid = lax.axis_index("s") * _NC + lax.axis_index("c")
        wbase = pl.multiple_of(wid * rows_pw, _SC_CH)
        pltpu.sync_copy(idx_hbm.at[pl.ds(wbase, rows_pw)], idx_v)
        bufs = ((rows0, sem_o0), (rows1, sem_o1))

        def body(j, carry):
            for b in range(2):
                c = j * 2 + b
                off = c * _SC_CH
                rows_v, sem_o = bufs[b]
                dst = out_hbm.at[pl.ds(wbase + off, _SC_CH)]

                @pl.when(j > 0)
                def _drain():
                    pltpu.make_async_copy(rows_v, dst, sem_o).wait()

                g = pltpu.make_async_copy(
                    table_hbm.at[idx_v.at[pl.ds(off, _SC_CH)]], rows_v, sem_g)
                g.start()
                g.wait()
                pltpu.make_async_copy(rows_v, dst, sem_o).start()
            return carry

        lax.fori_loop(0, nch // 2, body, 0)
        for b in range(2):
            rows_v, sem_o = bufs[b]
            pltpu.make_async_copy(
                rows_v, out_hbm.at[pl.ds(wbase, _SC_CH)], sem_o).wait()

    return gat


def _enc_body(x_ref, w1_ref, b1_ref, w2_ref, b2_ref, w3_ref, b3_ref,
              g_ref, be_ref, o_ref):
    h = jnp.maximum(_dot(x_ref[...], w1_ref[...]) + b1_ref[...], 0.0)
    h = jnp.maximum(_dot(h, w2_ref[...]) + b2_ref[...], 0.0)
    h = _dot(h, w3_ref[...]) + b3_ref[...]
    o_ref[...] = _ln(h, g_ref[...], be_ref[...])


def _mp_body(offs_ref, nh_ref, eh_any, gs_any, dst_any,
             ew1_ref, eb1_ref, ew2_ref, eb2_ref, ew3_ref, eb3_ref,
             eg_ref, ebe_ref,
             nw1_ref, nb1_ref, nw2_ref, nb2_ref, nw3_ref, nb3_ref,
             ng_ref, nbe_ref,
             o_node_ref, o_edge_any,
             ebuf, sbuf, dbuf, obuf, sem_in0, sem_in1, sem_o0, sem_o1):
    i = pl.program_id(0)
    start = offs_ref[i]
    end = offs_ref[i + 1]
    nck = (end - start + (_CH - 1)) // _CH
    npair = (nck + 1) // 2
    ntot = 2 * npair
    nh = nh_ref[...]
    cols = i * _NB + lax.broadcasted_iota(jnp.int32, (1, _NB), 1)
    sem_in = (sem_in0, sem_in1)
    sem_o = (sem_o0, sem_o1)
    ew1 = ew1_ref[...]

    def in_copies(c, b):
        base = start + c * _CH
        return (pltpu.make_async_copy(eh_any.at[pl.ds(base, _CH), :],
                                      ebuf.at[b], sem_in[b]),
                pltpu.make_async_copy(gs_any.at[pl.ds(base, _CH), :],
                                      sbuf.at[b], sem_in[b]),
                pltpu.make_async_copy(dst_any.at[pl.ds(base, _CH), :],
                                      dbuf.at[b], sem_in[b]))

    def prefetch(c, b):
        for cp in in_copies(c, b):
            cp.start()

    @pl.when(nck > 0)
    def _prime():
        prefetch(0, 0)
        prefetch(1, 1)

    def chunk(c, b, agg):
        base = start + c * _CH
        for cp in in_copies(c, b):
            cp.wait()
        e = ebuf[b]
        s = sbuf[b]
        dv = dbuf[b]

        @pl.when(c + 2 < ntot)
        def _next():
            prefetch(c + 2, b)

        ids = base + lax.broadcasted_iota(jnp.int32, (_CH, 1), 0)
        oh = jnp.where((dv == cols) & (ids < end), 1.0, 0.0)
        nd = _dot(oh, nh)
        h = (_dot(e, ew1[0:_H]) + _dot(s, ew1[_H:2 * _H])
             + _dot(nd, ew1[2 * _H:]) + eb1_ref[...])
        h = jnp.maximum(h, 0.0)
        h = jnp.maximum(_dot(h, ew2_ref[...]) + eb2_ref[...], 0.0)
        h = _dot(h, ew3_ref[...]) + eb3_ref[...]
        enew = _ln(h, eg_ref[...], ebe_ref[...]) + e
        out_cp = pltpu.make_async_copy(
            obuf.at[b], o_edge_any.at[pl.ds(base, _CH), :], sem_o[b])

        @pl.when(c >= 2)
        def _drain():
            out_cp.wait()

        obuf[b] = enew
        out_cp.start()
        return agg + lax.dot_general(oh, enew, (((0,), (0,)), ((), ())),
                                     preferred_element_type=jnp.float32)

    def pair(j, agg):
        agg = chunk(2 * j, 0, agg)
        return chunk(2 * j + 1, 1, agg)

    agg = lax.fori_loop(0, npair, pair, jnp.zeros((_NB, _H), jnp.float32))

    @pl.when(nck > 0)
    def _final_drain():
        for b in range(2):
            pltpu.make_async_copy(
                obuf.at[b], o_edge_any.at[pl.ds(start, _CH), :],
                sem_o[b]).wait()

    nw1 = nw1_ref[...]
    h = jnp.maximum(_dot(nh, nw1[:_H]) + _dot(agg, nw1[_H:]) + nb1_ref[...],
                    0.0)
    h = jnp.maximum(_dot(h, nw2_ref[...]) + nb2_ref[...], 0.0)
    h = _dot(h, nw3_ref[...]) + nb3_ref[...]
    o_node_ref[...] = _ln(h, ng_ref[...], nbe_ref[...]) + nh


def _dec_body(nh_ref, w1_ref, b1_ref, w2_ref, b2_ref, w3_ref, b3_ref,
              xr_ref, o_ref):
    h = jnp.maximum(_dot(nh_ref[...], w1_ref[...]) + b1_ref[...], 0.0)
    h = jnp.maximum(_dot(h, w2_ref[...]) + b2_ref[...], 0.0)
    o_ref[...] = _dot(h, w3_ref[...]) + b3_ref[...] + xr_ref[...]


def _wspec(shape):
    return pl.BlockSpec(shape, lambda i: tuple(0 for _ in shape))


def _enc_call(xin, W1, b1, W2, b2, W3, b3, g, be, blk):
    R = xin.shape[0]
    return pl.pallas_call(
        _enc_body,
        grid=(R // blk,),
        in_specs=[pl.BlockSpec((blk, xin.shape[1]), lambda i: (i, 0)),
                  _wspec(W1.shape), _wspec((1, _H)), _wspec(W2.shape),
                  _wspec((1, _H)), _wspec(W3.shape), _wspec((1, _H)),
                  _wspec((1, _H)), _wspec((1, _H))],
        out_specs=pl.BlockSpec((blk, _H), lambda i: (i, 0)),
        out_shape=jax.ShapeDtypeStruct((R, _H), jnp.float32),
    )(xin, W1, b1, W2, b2, W3, b3, g, be)


def _mp_call(node_h, offs, edge_h, g_src, dst2d, ew, nw):
    N_pad = node_h.shape[0]
    E_pad = edge_h.shape[0]
    return pl.pallas_call(
        _mp_body,
        grid=(N_pad // _NB,),
        in_specs=[pl.BlockSpec(memory_space=pltpu.SMEM),
                  pl.BlockSpec((_NB, _H), lambda i: (i, 0)),
                  pl.BlockSpec(memory_space=pl.ANY),
                  pl.BlockSpec(memory_space=pl.ANY),
                  pl.BlockSpec(memory_space=pl.ANY),
                  _wspec(ew[0].shape), _wspec((1, _H)), _wspec(ew[2].shape),
                  _wspec((1, _H)), _wspec(ew[4].shape), _wspec((1, _H)),
                  _wspec((1, _H)), _wspec((1, _H)),
                  _wspec(nw[0].shape), _wspec((1, _H)), _wspec(nw[2].shape),
                  _wspec((1, _H)), _wspec(nw[4].shape), _wspec((1, _H)),
                  _wspec((1, _H)), _wspec((1, _H))],
        out_specs=[pl.BlockSpec((_NB, _H), lambda i: (i, 0)),
                   pl.BlockSpec(memory_space=pl.ANY)],
        out_shape=[jax.ShapeDtypeStruct((N_pad, _H), jnp.float32),
                   jax.ShapeDtypeStruct((E_pad, _H), jnp.float32)],
        scratch_shapes=[pltpu.VMEM((2, _CH, _H), jnp.float32),
                        pltpu.VMEM((2, _CH, _H), jnp.float32),
                        pltpu.VMEM((2, _CH, 1), jnp.int32),
                        pltpu.VMEM((2, _CH, _H), jnp.float32),
                        pltpu.SemaphoreType.DMA,
                        pltpu.SemaphoreType.DMA,
                        pltpu.SemaphoreType.DMA,
                        pltpu.SemaphoreType.DMA],
    )(offs, node_h, edge_h, g_src, dst2d, *ew, *nw)


def _dec_call(node_h, W1, b1, W2, b2, W3, b3, xres):
    N_pad = node_h.shape[0]
    return pl.pallas_call(
        _dec_body,
        grid=(N_pad // _NB,),
        in_specs=[pl.BlockSpec((_NB, _H), lambda i: (i, 0)),
                  _wspec(W1.shape), _wspec((1, _H)), _wspec(W2.shape),
                  _wspec((1, _H)), _wspec(W3.shape), _wspec((1, _H)),
                  pl.BlockSpec((_NB, _H), lambda i: (i, 0))],
        out_specs=pl.BlockSpec((_NB, _H), lambda i: (i, 0)),
        out_shape=jax.ShapeDtypeStruct((N_pad, _H), jnp.float32),
    )(node_h, W1, b1, W2, b2, W3, b3, xres)


def _prep3(p, in_pad=None, out_pad=None):
    (W1, b1), (W2, b2), (W3, b3) = p["lin"]
    if in_pad is not None and W1.shape[0] < in_pad:
        W1 = jnp.zeros((in_pad, W1.shape[1]), jnp.float32).at[:W1.shape[0]].set(W1)
    if out_pad is not None and W3.shape[1] < out_pad:
        W3 = jnp.zeros((W3.shape[0], out_pad), jnp.float32).at[:, :W3.shape[1]].set(W3)
        b3 = jnp.zeros((out_pad,), jnp.float32).at[:b3.shape[0]].set(b3)
    ws = [W1, b1.reshape(1, -1), W2, b2.reshape(1, -1), W3, b3.reshape(1, -1)]
    if "ln" in p:
        g, be = p["ln"]
        ws += [g.reshape(1, -1), be.reshape(1, -1)]
    return ws


def kernel(x, edge_index, edge_attr, node_type, params):
    N, ndim = x.shape
    E, e_in = edge_attr.shape
    N_pad = _rup(N, _NB)
    E_pad = _rup(E + 2 * _CH, _NC * _NS * _SC_CH)
    nblk = N_pad // _NB

    src = edge_index[0].astype(jnp.int32)
    dst = edge_index[1].astype(jnp.int32)
    perm = jnp.argsort(dst)
    dst_s = dst[perm]
    src_s = src[perm]
    pad_e = E_pad - E
    dst_sp = jnp.concatenate([dst_s, jnp.full((pad_e,), N_pad - 1, jnp.int32)])
    src_sp = jnp.concatenate([src_s, jnp.zeros((pad_e,), jnp.int32)])
    offs = jnp.searchsorted(
        dst_s, jnp.arange(nblk + 1, dtype=jnp.int32) * _NB).astype(jnp.int32)
    dst2d = dst_sp.reshape(E_pad, 1)

    nt = jnp.squeeze(node_type).astype(jnp.int32)
    onehot = jax.nn.one_hot(nt, 2, dtype=jnp.float32)
    xin = (jnp.zeros((N_pad, _H), jnp.float32)
           .at[:N, :ndim].set(x).at[:N, ndim:ndim + 2].set(onehot))
    ein = jnp.zeros((E_pad, _H), jnp.float32).at[:E, :e_in].set(edge_attr[perm])
    xres = jnp.zeros((N_pad, _H), jnp.float32).at[:N, :ndim].set(x)

    enc_n = _prep3(params["node_enc"], in_pad=_H)
    enc_e = _prep3(params["edge_enc"], in_pad=_H)
    dec_w = _prep3(params["dec"], out_pad=_H)

    node_h = _enc_call(xin, *enc_n, blk=_NB)
    edge_h = _enc_call(ein, *enc_e, blk=_EB)

    gat_E = _make_gather(E_pad, _H)

    for blk in params["mp"]:
        g_src = gat_E(node_h, src_sp)
        node_h, edge_h = _mp_call(node_h, offs, edge_h, g_src, dst2d,
                                  _prep3(blk["edge"]), _prep3(blk["node"]))

    out = _dec_call(node_h, *dec_w, xres)
    return out[:N, :ndim]
